# baseline (speedup 1.0000x reference)
import functools

import jax
import jax.numpy as jnp
from jax import lax
from jax.experimental import pallas as pl
from jax.experimental.pallas import tpu as pltpu
from jax.experimental.pallas import tpu_sc as plsc

B = 16384
D = 64
NC = 2
NS = 16
NW = NC * NS
BW = B // NW
L = 16


def _sc_gather_partials(uidx, fidx, user_emb, user_bias, food_emb, food_bias):
    mesh = plsc.VectorSubcoreMesh(core_axis_name="c", subcore_axis_name="s")

    @functools.partial(
        pl.kernel,
        mesh=mesh,
        compiler_params=pltpu.CompilerParams(use_tc_tiling_on_sc=False),
        out_type=(
            jax.ShapeDtypeStruct((NW * L,), jnp.float32),
            jax.ShapeDtypeStruct((B,), jnp.float32),
            jax.ShapeDtypeStruct((B,), jnp.float32),
        ),
        scratch_types=[
            pltpu.VMEM((BW,), jnp.int32),
            pltpu.VMEM((BW,), jnp.int32),
            pltpu.VMEM((BW, D), jnp.float32),
            pltpu.VMEM((BW, D), jnp.float32),
            pltpu.VMEM((BW,), jnp.float32),
            pltpu.VMEM((BW,), jnp.float32),
            pltpu.VMEM((L,), jnp.float32),
            pltpu.SemaphoreType.DMA,
            pltpu.SemaphoreType.DMA,
            pltpu.SemaphoreType.DMA,
            pltpu.SemaphoreType.DMA,
        ],
    )
    def k(uidx_hbm, fidx_hbm, uemb_hbm, ubias_hbm, femb_hbm, fbias_hbm,
          part_hbm, ub_hbm, fb_hbm,
          uidx_v, fidx_v, urows_v, frows_v, ub_v, fb_v, part_v,
          sem_u, sem_f, sem_ub, sem_fb):
        wid = lax.axis_index("s") * NC + lax.axis_index("c")
        base = wid * BW
        pltpu.sync_copy(uidx_hbm.at[pl.ds(base, BW)], uidx_v)
        pltpu.sync_copy(fidx_hbm.at[pl.ds(base, BW)], fidx_v)
        cu = pltpu.async_copy(uemb_hbm.at[uidx_v], urows_v, sem_u)
        cf = pltpu.async_copy(femb_hbm.at[fidx_v], frows_v, sem_f)
        cub = pltpu.async_copy(ubias_hbm.at[uidx_v], ub_v, sem_ub)
        cfb = pltpu.async_copy(fbias_hbm.at[fidx_v], fb_v, sem_fb)
        cu.wait()
        cf.wait()

        zero = jnp.zeros((L,), jnp.float32)

        def row_body(i, accs):
            a0, a1, a2, a3 = accs
            a0 = a0 + urows_v[i, pl.ds(0 * L, L)] * frows_v[i, pl.ds(0 * L, L)]
            a1 = a1 + urows_v[i, pl.ds(1 * L, L)] * frows_v[i, pl.ds(1 * L, L)]
            a2 = a2 + urows_v[i, pl.ds(2 * L, L)] * frows_v[i, pl.ds(2 * L, L)]
            a3 = a3 + urows_v[i, pl.ds(3 * L, L)] * frows_v[i, pl.ds(3 * L, L)]
            return (a0, a1, a2, a3)

        a0, a1, a2, a3 = lax.fori_loop(0, BW, row_body, (zero, zero, zero, zero))
        part_v[...] = (a0 + a1) + (a2 + a3)
        pltpu.sync_copy(part_v, part_hbm.at[pl.ds(wid * L, L)])
        cub.wait()
        cfb.wait()
        pltpu.sync_copy(ub_v, ub_hbm.at[pl.ds(base, BW)])
        pltpu.sync_copy(fb_v, fb_hbm.at[pl.ds(base, BW)])

    return k(uidx, fidx, user_emb, user_bias, food_emb, food_bias)


def _tc_finish(part, ub, fb):
    def body(p_ref, u_ref, f_ref, o_ref):
        s = jnp.sum(p_ref[...])
        o_ref[...] = jax.nn.sigmoid(u_ref[...] + f_ref[...] + s)

    return pl.pallas_call(
        body,
        out_shape=jax.ShapeDtypeStruct((128, 128), jnp.float32),
    )(part.reshape(4, 128), ub.reshape(128, 128), fb.reshape(128, 128))


def kernel(inputs, user_emb, user_bias, food_emb, food_bias):
    uidx = inputs[:, 0]
    fidx = inputs[:, 1]
    part, ub, fb = _sc_gather_partials(
        uidx, fidx, user_emb, user_bias.reshape(-1), food_emb,
        food_bias.reshape(-1))
    return _tc_finish(part, ub, fb).reshape(B, 1)
